# baseline (speedup 1.0000x reference)
import functools

import jax
import jax.numpy as jnp
from jax import lax
from jax.experimental import pallas as pl
from jax.experimental.pallas import tpu as pltpu
from jax.experimental.pallas import tpu_sc as plsc

REC_N = 10000; REC_E = 40000; LIG_N = 640; LIG_E = 2560; B = 16; D = 32
NW = 32


def _sc_mesh():
    return plsc.VectorSubcoreMesh(core_axis_name="c", subcore_axis_name="s")


def _node_kernel(cat_ref, scal_ref, t0_ref, t1_ref, wp_ref, b_ref, out_ref, *, n, s):
    c0 = cat_ref[:, 0:1]
    c1 = cat_ref[:, 1:2]
    iot = lax.broadcasted_iota(jnp.int32, (n, 32), 1)
    o0 = (iot == c0).astype(jnp.float32)
    o1 = (iot == c1).astype(jnp.float32)
    e0 = jnp.dot(o0, t0_ref[...], preferred_element_type=jnp.float32, precision=lax.Precision.HIGHEST)
    e1 = jnp.dot(o1, t1_ref[...], preferred_element_type=jnp.float32, precision=lax.Precision.HIGHEST)
    h = (jnp.dot(e0, wp_ref[0:32, :], preferred_element_type=jnp.float32, precision=lax.Precision.HIGHEST)
         + jnp.dot(e1, wp_ref[32:64, :], preferred_element_type=jnp.float32, precision=lax.Precision.HIGHEST)
         + jnp.dot(scal_ref[...], wp_ref[64:64 + s, :], preferred_element_type=jnp.float32, precision=lax.Precision.HIGHEST)
         + b_ref[...])
    out_ref[...] = jnp.maximum(h, 0.0)


def _node_embed_proj(cat, scal, tables, wp, b):
    n, s = scal.shape
    return pl.pallas_call(
        functools.partial(_node_kernel, n=n, s=s),
        out_shape=jax.ShapeDtypeStruct((n, D), jnp.float32),
    )(cat, scal, tables[0], tables[1], wp, b.reshape(1, D))


def _edge_kernel(cat_ref, scal_ref, te_ref, w1_ref, b1_ref, w2_ref, b2_ref,
                 out_ref, *, te, e_real):
    pid = pl.program_id(0)
    oh = (lax.broadcasted_iota(jnp.int32, (te, 8), 1) == cat_ref[...]).astype(jnp.float32)
    ef16 = jnp.dot(oh, te_ref[...], preferred_element_type=jnp.float32, precision=lax.Precision.HIGHEST)
    a = (jnp.dot(ef16, w1_ref[0:16, :], preferred_element_type=jnp.float32, precision=lax.Precision.HIGHEST)
         + jnp.dot(scal_ref[...], w1_ref[16:20, :], preferred_element_type=jnp.float32, precision=lax.Precision.HIGHEST)
         + b1_ref[...])
    a = jnp.maximum(a, 0.0)
    ew = jnp.dot(a, w2_ref[...], preferred_element_type=jnp.float32, precision=lax.Precision.HIGHEST) + b2_ref[...]
    egl = pid * te + lax.broadcasted_iota(jnp.int32, (te, 1), 0)
    out_ref[...] = jnp.where(egl < e_real, ew, 0.0)


def _edge_weights(cat_p, scal_p, tbl, w1, b1, w2, b2, e_real):
    e_pad = cat_p.shape[0]
    te = 512
    grid = e_pad // te
    return pl.pallas_call(
        functools.partial(_edge_kernel, te=te, e_real=e_real),
        grid=(grid,),
        in_specs=[
            pl.BlockSpec((te, 1), lambda i: (i, 0)),
            pl.BlockSpec((te, 4), lambda i: (i, 0)),
            pl.BlockSpec((8, 16), lambda i: (0, 0)),
            pl.BlockSpec((20, 128), lambda i: (0, 0)),
            pl.BlockSpec((1, 128), lambda i: (0, 0)),
            pl.BlockSpec((128, D * D), lambda i: (0, 0)),
            pl.BlockSpec((1, D * D), lambda i: (0, 0)),
        ],
        out_specs=pl.BlockSpec((te, D * D), lambda i: (i, 0)),
        out_shape=jax.ShapeDtypeStruct((e_pad, D * D), jnp.float32),
    )(cat_p, scal_p, tbl[0], w1, b1.reshape(1, 128), w2, b2.reshape(1, D * D))


def _msg_kernel(h_ref, ew_ref, out_ref, *, tm):
    h3 = h_ref[...][:, :, None]
    out_ref[...] = jnp.sum(ew_ref[...] * h3, axis=1)


def _messages(hsrc, ew3):
    e_pad = hsrc.shape[0]
    tm = 512
    return pl.pallas_call(
        functools.partial(_msg_kernel, tm=tm),
        grid=(e_pad // tm,),
        in_specs=[
            pl.BlockSpec((tm, D), lambda i: (i, 0)),
            pl.BlockSpec((tm, D, D), lambda i: (i, 0, 0)),
        ],
        out_specs=pl.BlockSpec((tm, D), lambda i: (i, 0)),
        out_shape=jax.ShapeDtypeStruct((e_pad, D), jnp.float32),
    )(hsrc, ew3)


def _gru_kernel(agg_ref, hid_ref, nnb_ref, wi_ref, bi_ref, wh_ref, bh_ref, out_ref):
    agg = agg_ref[0] + agg_ref[1]
    x = jnp.maximum(agg + nnb_ref[...], 0.0)
    h = hid_ref[...]
    gi = jnp.dot(x, wi_ref[...], preferred_element_type=jnp.float32, precision=lax.Precision.HIGHEST) + bi_ref[...]
    gh = jnp.dot(h, wh_ref[...], preferred_element_type=jnp.float32, precision=lax.Precision.HIGHEST) + bh_ref[...]
    r = jax.nn.sigmoid(gi[:, 0:D] + gh[:, 0:D])
    z = jax.nn.sigmoid(gi[:, D:2 * D] + gh[:, D:2 * D])
    nn = jnp.tanh(gi[:, 2 * D:3 * D] + r * gh[:, 2 * D:3 * D])
    out_ref[...] = (1.0 - z) * nn + z * h


def _gru(agg2, hidden, nnb, wi, bi, wh, bh):
    n = hidden.shape[0]
    return pl.pallas_call(
        _gru_kernel,
        out_shape=jax.ShapeDtypeStruct((n, D), jnp.float32),
    )(agg2, hidden, nnb.reshape(1, D), wi, bi.reshape(1, 3 * D), wh, bh.reshape(1, 3 * D))


def _readout_kernel(h_ref, grow_ref, gcol_ref, w_ref, b_ref, out_ref, *, n):
    h = h_ref[...]
    w = jax.nn.sigmoid(jnp.dot(h, w_ref[...], preferred_element_type=jnp.float32, precision=lax.Precision.HIGHEST) + b_ref[...])
    wh = w * h
    onehot = (grow_ref[...] == lax.broadcasted_iota(jnp.int32, (B, n), 0)).astype(jnp.float32)
    ws = jnp.dot(onehot, wh, preferred_element_type=jnp.float32, precision=lax.Precision.HIGHEST)
    gcol = gcol_ref[...]
    mxs = []
    for bb in range(B):
        hm = jnp.where(gcol == bb, h, -jnp.inf)
        mxs.append(jnp.max(hm, axis=0, keepdims=True))
    mx = jnp.concatenate(mxs, axis=0)
    out_ref[...] = jnp.concatenate([ws, mx], axis=1)


def _readout(h, gids, w, b):
    n = h.shape[0]
    return pl.pallas_call(
        functools.partial(_readout_kernel, n=n),
        out_shape=jax.ShapeDtypeStruct((B, 2 * D), jnp.float32),
    )(h, gids.reshape(1, n), gids.reshape(n, 1), w, b.reshape(1, 1))


def _head_kernel(rr_ref, lr_ref, w1_ref, b1_ref, g1_ref, bb1_ref,
                 w2_ref, b2_ref, g2_ref, bb2_ref, ow_ref, ob_ref, out_ref):
    rr = rr_ref[...]
    lr = lr_ref[...]
    C = jnp.dot(lr, w1_ref[...], preferred_element_type=jnp.float32, precision=lax.Precision.HIGHEST)
    y = jnp.zeros((B, 256), jnp.float32)
    for i in range(64):
        y = y + rr[:, i:i + 1] * C[:, i * 256:(i + 1) * 256]
    y = y + b1_ref[...]
    mu = jnp.mean(y, axis=-1, keepdims=True)
    v = jnp.mean((y - mu) * (y - mu), axis=-1, keepdims=True)
    y = (y - mu) * lax.rsqrt(v + 1e-5) * g1_ref[...] + bb1_ref[...]
    y = jnp.where(y > 0, y, 0.01 * y)
    y = jnp.dot(y, w2_ref[...], preferred_element_type=jnp.float32, precision=lax.Precision.HIGHEST) + b2_ref[...]
    mu = jnp.mean(y, axis=-1, keepdims=True)
    v = jnp.mean((y - mu) * (y - mu), axis=-1, keepdims=True)
    y = (y - mu) * lax.rsqrt(v + 1e-5) * g2_ref[...] + bb2_ref[...]
    y = jnp.where(y > 0, y, 0.01 * y)
    out_ref[...] = jnp.dot(y, ow_ref[...], preferred_element_type=jnp.float32, precision=lax.Precision.HIGHEST) + ob_ref[...]


def _head(rr, lr, p):
    w1v = p['mlp_W1'].reshape(64, 64, 256).transpose(1, 0, 2).reshape(64, 64 * 256)
    out = pl.pallas_call(
        _head_kernel,
        out_shape=jax.ShapeDtypeStruct((B, 1), jnp.float32),
    )(rr, lr, w1v, p['mlp_b1'].reshape(1, 256), p['ln1_g'].reshape(1, 256),
      p['ln1_b'].reshape(1, 256), p['mlp_W2'], p['mlp_b2'].reshape(1, 64),
      p['ln2_g'].reshape(1, 64), p['ln2_b'].reshape(1, 64), p['out_W'],
      p['out_b'].reshape(1, 1))
    return out[:, 0]


def _sc_gather(h, src3):
    nw, ch, ck = src3.shape
    eb = ch * ck
    e_pad = nw * eb

    @functools.partial(
        pl.kernel,
        out_type=jax.ShapeDtypeStruct((e_pad, D), jnp.float32),
        mesh=_sc_mesh(),
        compiler_params=pltpu.CompilerParams(use_tc_tiling_on_sc=False),
        scratch_types=[
            pltpu.VMEM((ch, ck), jnp.int32),
            pltpu.VMEM((eb, D), jnp.float32),
            pltpu.SemaphoreType.DMA,
        ],
    )
    def gk(h_hbm, src_hbm, out_hbm, idx_v, rows_v, sem):
        wid = lax.axis_index("s") * 2 + lax.axis_index("c")
        base = wid * eb
        pltpu.sync_copy(src_hbm.at[wid], idx_v)
        for j in range(ch):
            pltpu.async_copy(h_hbm.at[idx_v.at[j]], rows_v.at[pl.ds(j * ck, ck)], sem)
        for j in range(ch):
            pltpu.make_async_copy(h_hbm.at[idx_v.at[j]],
                                  rows_v.at[pl.ds(j * ck, ck)], sem).wait()
        pltpu.sync_copy(rows_v, out_hbm.at[pl.ds(base, eb)])

    return gk(h, src3)


def _sc_scatter(msg, dst3, zeros_n):
    nw, ch, ck = dst3.shape
    eb = ch * ck
    n = zeros_n.shape[0]
    nr = n // 16

    @functools.partial(
        pl.kernel,
        out_type=jax.ShapeDtypeStruct((2, n, D), jnp.float32),
        mesh=_sc_mesh(),
        compiler_params=pltpu.CompilerParams(use_tc_tiling_on_sc=False),
        scratch_types=[
            pltpu.VMEM((ch, ck), jnp.int32),
            pltpu.VMEM((eb, D), jnp.float32),
            pltpu.VMEM_SHARED((n, D), jnp.float32),
        ],
    )
    def sk(msg_hbm, dst_hbm, zeros_hbm, out_hbm, idx_v, msg_v, acc_sh):
        c = lax.axis_index("c")
        s = lax.axis_index("s")
        wid = s * 2 + c
        base = wid * eb
        pltpu.sync_copy(dst_hbm.at[wid], idx_v)
        pltpu.sync_copy(msg_hbm.at[pl.ds(base, eb)], msg_v)
        pltpu.sync_copy(zeros_hbm.at[pl.ds(s * nr, nr)], acc_sh.at[pl.ds(s * nr, nr)])
        plsc.subcore_barrier()
        for j in range(ch):
            pltpu.sync_copy(msg_v.at[pl.ds(j * ck, ck)], acc_sh.at[idx_v.at[j]], add=True)
        plsc.subcore_barrier()
        pltpu.sync_copy(acc_sh.at[pl.ds(s * nr, nr)], out_hbm.at[c, pl.ds(s * nr, nr)])

    return sk(msg, dst3, zeros_n)


def _mpnn(p, pre, cat, scal, ecat_p, escal_p, src_p, dst_p, n_nodes, e_real, n_layers):
    h = _node_embed_proj(cat, scal, p[pre + '_node_emb'], p[pre + '_proj_W'], p[pre + '_proj_b'])
    ew = _edge_weights(ecat_p, escal_p, p[pre + '_edge_emb'], p[pre + '_enW1'],
                       p[pre + '_enb1'], p[pre + '_enW2'], p[pre + '_enb2'], e_real)
    ew3 = ew.reshape(-1, D, D)
    zeros_n = jnp.zeros((n_nodes, D), jnp.float32)
    eb = src_p.shape[0] // NW
    ck = 128 if eb % 128 == 0 else eb
    src3 = src_p.reshape(NW, eb // ck, ck)
    dst3 = dst_p.reshape(NW, eb // ck, ck)
    hidden = h
    for _ in range(n_layers):
        hsrc = _sc_gather(h, src3)
        msg = _messages(hsrc, ew3)
        agg2 = _sc_scatter(msg, dst3, zeros_n)
        hidden = _gru(agg2, hidden, p[pre + '_nn_b'], p[pre + '_Wi'],
                      p[pre + '_bi'], p[pre + '_Wh'], p[pre + '_bh'])
        h = hidden
    return h


def _pad_edges(arr, e_pad):
    e = arr.shape[0]
    if e == e_pad:
        return arr
    pad = [(0, e_pad - e)] + [(0, 0)] * (arr.ndim - 1)
    return jnp.pad(arr, pad)


@jax.jit
def kernel(rec_node_cat, rec_node_scal, rec_edge_cat, rec_edge_scal, rec_edge_src, rec_edge_dst, rec_graph_ids, lig_node_cat, lig_node_scal, lig_edge_cat, lig_edge_scal, lig_edge_src, lig_edge_dst, lig_graph_ids, params):
    p = params
    rec_ep = 40960
    lig_ep = LIG_E

    rec_out = _mpnn(p, 'rec', rec_node_cat, rec_node_scal,
                    _pad_edges(rec_edge_cat, rec_ep), _pad_edges(rec_edge_scal, rec_ep),
                    _pad_edges(rec_edge_src, rec_ep), _pad_edges(rec_edge_dst, rec_ep),
                    REC_N, REC_E, 2)
    lig_out = _mpnn(p, 'lig', lig_node_cat, lig_node_scal,
                    lig_edge_cat, lig_edge_scal, lig_edge_src, lig_edge_dst,
                    LIG_N, LIG_E, 3)
    rr = _readout(rec_out, rec_graph_ids, p['rec_rw_W'], p['rec_rw_b'])
    lr = _readout(lig_out, lig_graph_ids, p['lig_rw_W'], p['lig_rw_b'])
    return _head(rr, lr, p)


# speedup vs baseline: 1.3612x; 1.3612x over previous
import functools

import jax
import jax.numpy as jnp
from jax import lax
from jax.experimental import pallas as pl
from jax.experimental.pallas import tpu as pltpu
from jax.experimental.pallas import tpu_sc as plsc

REC_N = 10000; REC_E = 40000; LIG_N = 640; LIG_E = 2560; B = 16; D = 32
NW = 32


def _sc_mesh():
    return plsc.VectorSubcoreMesh(core_axis_name="c", subcore_axis_name="s")


def _node_kernel(cat_ref, scal_ref, t0_ref, t1_ref, wp_ref, b_ref, out_ref, *, n, s):
    c0 = cat_ref[:, 0:1]
    c1 = cat_ref[:, 1:2]
    iot = lax.broadcasted_iota(jnp.int32, (n, 32), 1)
    o0 = (iot == c0).astype(jnp.float32)
    o1 = (iot == c1).astype(jnp.float32)
    e0 = jnp.dot(o0, t0_ref[...], preferred_element_type=jnp.float32,
                 precision=lax.Precision.HIGHEST)
    e1 = jnp.dot(o1, t1_ref[...], preferred_element_type=jnp.float32,
                 precision=lax.Precision.HIGHEST)
    hp = lax.Precision.HIGHEST
    h = (jnp.dot(e0, wp_ref[0:32, :], preferred_element_type=jnp.float32, precision=hp)
         + jnp.dot(e1, wp_ref[32:64, :], preferred_element_type=jnp.float32, precision=hp)
         + jnp.dot(scal_ref[...], wp_ref[64:64 + s, :], preferred_element_type=jnp.float32, precision=hp)
         + b_ref[...])
    out_ref[...] = jnp.maximum(h, 0.0)


def _node_embed_proj(cat, scal, tables, wp, b):
    n, s = scal.shape
    return pl.pallas_call(
        functools.partial(_node_kernel, n=n, s=s),
        out_shape=jax.ShapeDtypeStruct((n, D), jnp.float32),
    )(cat, scal, tables[0], tables[1], wp, b.reshape(1, D))


def _edge_kernel(cat_ref, scal_ref, te_ref, w1_ref, b1_ref, w2_ref, b2_ref,
                 out_ref, *, te, e_real):
    pid = pl.program_id(0)
    oh = (lax.broadcasted_iota(jnp.int32, (te, 8), 1) == cat_ref[...]).astype(jnp.float32)
    ef16 = jnp.dot(oh, te_ref[...], preferred_element_type=jnp.float32,
                   precision=lax.Precision.HIGHEST)
    hp = lax.Precision.HIGHEST
    a = (jnp.dot(ef16, w1_ref[0:16, :], preferred_element_type=jnp.float32, precision=hp)
         + jnp.dot(scal_ref[...], w1_ref[16:20, :], preferred_element_type=jnp.float32, precision=hp)
         + b1_ref[...])
    a = jnp.maximum(a, 0.0)
    ew = jnp.dot(a, w2_ref[...], preferred_element_type=jnp.float32,
                 precision=lax.Precision.HIGHEST) + b2_ref[...]
    egl = pid * te + lax.broadcasted_iota(jnp.int32, (te, 1), 0)
    out_ref[...] = jnp.where(egl < e_real, ew, 0.0)


def _edge_weights(cat_p, scal_p, tbl, w1, b1, w2, b2, e_real):
    e_pad = cat_p.shape[0]
    te = 512
    grid = e_pad // te
    return pl.pallas_call(
        functools.partial(_edge_kernel, te=te, e_real=e_real),
        grid=(grid,),
        in_specs=[
            pl.BlockSpec((te, 1), lambda i: (i, 0)),
            pl.BlockSpec((te, 4), lambda i: (i, 0)),
            pl.BlockSpec((8, 16), lambda i: (0, 0)),
            pl.BlockSpec((20, 128), lambda i: (0, 0)),
            pl.BlockSpec((1, 128), lambda i: (0, 0)),
            pl.BlockSpec((128, D * D), lambda i: (0, 0)),
            pl.BlockSpec((1, D * D), lambda i: (0, 0)),
        ],
        out_specs=pl.BlockSpec((te, D * D), lambda i: (i, 0)),
        out_shape=jax.ShapeDtypeStruct((e_pad, D * D), jnp.float32),
    )(cat_p, scal_p, tbl[0], w1, b1.reshape(1, 128), w2, b2.reshape(1, D * D))


def _msg_kernel(h_ref, ew_ref, r_ref, s_ref, out_ref, *, tm):
    hp = lax.Precision.HIGHEST
    h4 = jnp.dot(h_ref[...], r_ref[...], preferred_element_type=jnp.float32, precision=hp)
    p = h4 * ew_ref[...]
    out_ref[...] = jnp.dot(p, s_ref[...], preferred_element_type=jnp.float32, precision=hp)


def _messages(hsrc, ew2d, rmat, smat):
    e_pad = hsrc.shape[0]
    tm = 512
    return pl.pallas_call(
        functools.partial(_msg_kernel, tm=tm),
        grid=(e_pad // tm,),
        in_specs=[
            pl.BlockSpec((tm, D), lambda i: (i, 0)),
            pl.BlockSpec((tm, D * D), lambda i: (i, 0)),
            pl.BlockSpec((D, D * D), lambda i: (0, 0)),
            pl.BlockSpec((D * D, D), lambda i: (0, 0)),
        ],
        out_specs=pl.BlockSpec((tm, D), lambda i: (i, 0)),
        out_shape=jax.ShapeDtypeStruct((e_pad, D), jnp.float32),
    )(hsrc, ew2d, rmat, smat)


def _gru_kernel(agg_ref, hid_ref, nnb_ref, wi_ref, bi_ref, wh_ref, bh_ref, out_ref):
    agg = agg_ref[0] + agg_ref[1]
    x = jnp.maximum(agg + nnb_ref[...], 0.0)
    h = hid_ref[...]
    hp = lax.Precision.HIGHEST
    gi = jnp.dot(x, wi_ref[...], preferred_element_type=jnp.float32, precision=hp) + bi_ref[...]
    gh = jnp.dot(h, wh_ref[...], preferred_element_type=jnp.float32, precision=hp) + bh_ref[...]
    r = jax.nn.sigmoid(gi[:, 0:D] + gh[:, 0:D])
    z = jax.nn.sigmoid(gi[:, D:2 * D] + gh[:, D:2 * D])
    nn = jnp.tanh(gi[:, 2 * D:3 * D] + r * gh[:, 2 * D:3 * D])
    out_ref[...] = (1.0 - z) * nn + z * h


def _gru(agg2, hidden, nnb, wi, bi, wh, bh):
    n = hidden.shape[0]
    return pl.pallas_call(
        _gru_kernel,
        out_shape=jax.ShapeDtypeStruct((n, D), jnp.float32),
    )(agg2, hidden, nnb.reshape(1, D), wi, bi.reshape(1, 3 * D), wh, bh.reshape(1, 3 * D))


def _readout_kernel(h_ref, grow_ref, gcol_ref, w_ref, b_ref, out_ref, *, n):
    h = h_ref[...]
    w = jax.nn.sigmoid(jnp.dot(h, w_ref[...], preferred_element_type=jnp.float32,
                               precision=lax.Precision.HIGHEST) + b_ref[...])
    wh = w * h
    onehot = (grow_ref[...] == lax.broadcasted_iota(jnp.int32, (B, n), 0)).astype(jnp.float32)
    ws = jnp.dot(onehot, wh, preferred_element_type=jnp.float32,
                 precision=lax.Precision.HIGHEST)
    gcol = gcol_ref[...]
    mxs = []
    for bb in range(B):
        hm = jnp.where(gcol == bb, h, -jnp.inf)
        mxs.append(jnp.max(hm, axis=0, keepdims=True))
    mx = jnp.concatenate(mxs, axis=0)
    out_ref[...] = jnp.concatenate([ws, mx], axis=1)


def _readout(h, gids, w, b):
    n = h.shape[0]
    return pl.pallas_call(
        functools.partial(_readout_kernel, n=n),
        out_shape=jax.ShapeDtypeStruct((B, 2 * D), jnp.float32),
    )(h, gids.reshape(1, n), gids.reshape(n, 1), w, b.reshape(1, 1))


def _head_kernel(rr_ref, lr_ref, w1_ref, b1_ref, g1_ref, bb1_ref,
                 w2_ref, b2_ref, g2_ref, bb2_ref, ow_ref, ob_ref, out_ref):
    rr = rr_ref[...]
    lr = lr_ref[...]
    C = jnp.dot(lr, w1_ref[...], preferred_element_type=jnp.float32,
                precision=lax.Precision.HIGHEST)
    y = jnp.zeros((B, 256), jnp.float32)
    for i in range(64):
        y = y + rr[:, i:i + 1] * C[:, i * 256:(i + 1) * 256]
    y = y + b1_ref[...]
    mu = jnp.mean(y, axis=-1, keepdims=True)
    v = jnp.mean((y - mu) * (y - mu), axis=-1, keepdims=True)
    y = (y - mu) * lax.rsqrt(v + 1e-5) * g1_ref[...] + bb1_ref[...]
    y = jnp.where(y > 0, y, 0.01 * y)
    y = jnp.dot(y, w2_ref[...], preferred_element_type=jnp.float32,
                precision=lax.Precision.HIGHEST) + b2_ref[...]
    mu = jnp.mean(y, axis=-1, keepdims=True)
    v = jnp.mean((y - mu) * (y - mu), axis=-1, keepdims=True)
    y = (y - mu) * lax.rsqrt(v + 1e-5) * g2_ref[...] + bb2_ref[...]
    y = jnp.where(y > 0, y, 0.01 * y)
    out_ref[...] = jnp.dot(y, ow_ref[...], preferred_element_type=jnp.float32,
                           precision=lax.Precision.HIGHEST) + ob_ref[...]


def _head(rr, lr, p):
    w1v = p['mlp_W1'].reshape(64, 64, 256).transpose(1, 0, 2).reshape(64, 64 * 256)
    out = pl.pallas_call(
        _head_kernel,
        out_shape=jax.ShapeDtypeStruct((B, 1), jnp.float32),
    )(rr, lr, w1v, p['mlp_b1'].reshape(1, 256), p['ln1_g'].reshape(1, 256),
      p['ln1_b'].reshape(1, 256), p['mlp_W2'], p['mlp_b2'].reshape(1, 64),
      p['ln2_g'].reshape(1, 64), p['ln2_b'].reshape(1, 64), p['out_W'],
      p['out_b'].reshape(1, 1))
    return out[:, 0]


def _sc_gather(h, src3):
    nw, ch, ck = src3.shape
    eb = ch * ck
    e_pad = nw * eb

    @functools.partial(
        pl.kernel,
        out_type=jax.ShapeDtypeStruct((e_pad, D), jnp.float32),
        mesh=_sc_mesh(),
        compiler_params=pltpu.CompilerParams(use_tc_tiling_on_sc=False),
        scratch_types=[
            pltpu.VMEM((ch, ck), jnp.int32),
            pltpu.VMEM((eb, D), jnp.float32),
            pltpu.SemaphoreType.DMA,
        ],
    )
    def gk(h_hbm, src_hbm, out_hbm, idx_v, rows_v, sem):
        wid = lax.axis_index("s") * 2 + lax.axis_index("c")
        base = wid * eb
        pltpu.sync_copy(src_hbm.at[wid], idx_v)
        for j in range(ch):
            pltpu.async_copy(h_hbm.at[idx_v.at[j]], rows_v.at[pl.ds(j * ck, ck)], sem)
        for j in range(ch):
            pltpu.make_async_copy(h_hbm.at[idx_v.at[j]],
                                  rows_v.at[pl.ds(j * ck, ck)], sem).wait()
        pltpu.sync_copy(rows_v, out_hbm.at[pl.ds(base, eb)])

    return gk(h, src3)


def _sc_scatter(msg, dst3, zeros_n):
    nw, ch, ck = dst3.shape
    eb = ch * ck
    n = zeros_n.shape[0]
    nr = n // 16

    @functools.partial(
        pl.kernel,
        out_type=jax.ShapeDtypeStruct((2, n, D), jnp.float32),
        mesh=_sc_mesh(),
        compiler_params=pltpu.CompilerParams(use_tc_tiling_on_sc=False),
        scratch_types=[
            pltpu.VMEM((ch, ck), jnp.int32),
            pltpu.VMEM((eb, D), jnp.float32),
            pltpu.VMEM_SHARED((n, D), jnp.float32),
        ],
    )
    def sk(msg_hbm, dst_hbm, zeros_hbm, out_hbm, idx_v, msg_v, acc_sh):
        c = lax.axis_index("c")
        s = lax.axis_index("s")
        wid = s * 2 + c
        base = wid * eb
        pltpu.sync_copy(dst_hbm.at[wid], idx_v)
        pltpu.sync_copy(msg_hbm.at[pl.ds(base, eb)], msg_v)
        pltpu.sync_copy(zeros_hbm.at[pl.ds(s * nr, nr)], acc_sh.at[pl.ds(s * nr, nr)])
        plsc.subcore_barrier()
        for j in range(ch):
            pltpu.sync_copy(msg_v.at[pl.ds(j * ck, ck)], acc_sh.at[idx_v.at[j]], add=True)
        plsc.subcore_barrier()
        pltpu.sync_copy(acc_sh.at[pl.ds(s * nr, nr)], out_hbm.at[c, pl.ds(s * nr, nr)])

    return sk(msg, dst3, zeros_n)


def _mpnn(p, pre, cat, scal, ecat_p, escal_p, src_p, dst_p, n_nodes, e_real, n_layers):
    h = _node_embed_proj(cat, scal, p[pre + '_node_emb'], p[pre + '_proj_W'], p[pre + '_proj_b'])
    ew = _edge_weights(ecat_p, escal_p, p[pre + '_edge_emb'], p[pre + '_enW1'],
                       p[pre + '_enb1'], p[pre + '_enW2'], p[pre + '_enb2'], e_real)
    eye = jnp.eye(D, dtype=jnp.float32)
    rmat = jnp.kron(eye, jnp.ones((1, D), jnp.float32))
    smat = jnp.tile(eye, (D, 1))
    zeros_n = jnp.zeros((n_nodes, D), jnp.float32)
    eb = src_p.shape[0] // NW
    ck = 128 if eb % 128 == 0 else eb
    src3 = src_p.reshape(NW, eb // ck, ck)
    dst3 = dst_p.reshape(NW, eb // ck, ck)
    hidden = h
    for _ in range(n_layers):
        hsrc = _sc_gather(h, src3)
        msg = _messages(hsrc, ew, rmat, smat)
        agg2 = _sc_scatter(msg, dst3, zeros_n)
        hidden = _gru(agg2, hidden, p[pre + '_nn_b'], p[pre + '_Wi'],
                      p[pre + '_bi'], p[pre + '_Wh'], p[pre + '_bh'])
        h = hidden
    return h


def _pad_edges(arr, e_pad):
    e = arr.shape[0]
    if e == e_pad:
        return arr
    pad = [(0, e_pad - e)] + [(0, 0)] * (arr.ndim - 1)
    return jnp.pad(arr, pad)


@jax.jit
def kernel(rec_node_cat, rec_node_scal, rec_edge_cat, rec_edge_scal, rec_edge_src, rec_edge_dst, rec_graph_ids, lig_node_cat, lig_node_scal, lig_edge_cat, lig_edge_scal, lig_edge_src, lig_edge_dst, lig_graph_ids, params):
    p = params
    rec_ep = 40960
    lig_ep = LIG_E

    rec_out = _mpnn(p, 'rec', rec_node_cat, rec_node_scal,
                    _pad_edges(rec_edge_cat, rec_ep), _pad_edges(rec_edge_scal, rec_ep),
                    _pad_edges(rec_edge_src, rec_ep), _pad_edges(rec_edge_dst, rec_ep),
                    REC_N, REC_E, 2)
    lig_out = _mpnn(p, 'lig', lig_node_cat, lig_node_scal,
                    lig_edge_cat, lig_edge_scal, lig_edge_src, lig_edge_dst,
                    LIG_N, LIG_E, 3)
    rr = _readout(rec_out, rec_graph_ids, p['rec_rw_W'], p['rec_rw_b'])
    lr = _readout(lig_out, lig_graph_ids, p['lig_rw_W'], p['lig_rw_b'])
    return _head(rr, lr, p)


# speedup vs baseline: 2.2765x; 1.6724x over previous
import functools

import jax
import jax.numpy as jnp
from jax import lax
from jax.experimental import pallas as pl
from jax.experimental.pallas import tpu as pltpu
from jax.experimental.pallas import tpu_sc as plsc

REC_N = 10000; REC_E = 40000; LIG_N = 640; LIG_E = 2560; B = 16; D = 32
NW = 32


def _sc_mesh():
    return plsc.VectorSubcoreMesh(core_axis_name="c", subcore_axis_name="s")


def _node_kernel(cat_ref, scal_ref, t0_ref, t1_ref, wp_ref, b_ref, out_ref, *, n, s):
    c0 = cat_ref[:, 0:1]
    c1 = cat_ref[:, 1:2]
    iot = lax.broadcasted_iota(jnp.int32, (n, 32), 1)
    o0 = (iot == c0).astype(jnp.float32)
    o1 = (iot == c1).astype(jnp.float32)
    e0 = jnp.dot(o0, t0_ref[...], preferred_element_type=jnp.float32,
                 precision=lax.Precision.HIGHEST)
    e1 = jnp.dot(o1, t1_ref[...], preferred_element_type=jnp.float32,
                 precision=lax.Precision.HIGHEST)
    hp = lax.Precision.HIGHEST
    h = (jnp.dot(e0, wp_ref[0:32, :], preferred_element_type=jnp.float32, precision=hp)
         + jnp.dot(e1, wp_ref[32:64, :], preferred_element_type=jnp.float32, precision=hp)
         + jnp.dot(scal_ref[...], wp_ref[64:64 + s, :], preferred_element_type=jnp.float32, precision=hp)
         + b_ref[...])
    out_ref[...] = jnp.maximum(h, 0.0)


def _node_embed_proj(cat, scal, tables, wp, b):
    n, s = scal.shape
    return pl.pallas_call(
        functools.partial(_node_kernel, n=n, s=s),
        out_shape=jax.ShapeDtypeStruct((n, D), jnp.float32),
    )(cat, scal, tables[0], tables[1], wp, b.reshape(1, D))


def _edge_kernel(cat_ref, scal_ref, te_ref, w1_ref, b1_ref, w2h_ref, w2l_ref,
                 b2_ref, out_ref, *, te, e_real):
    pid = pl.program_id(0)
    oh = (lax.broadcasted_iota(jnp.int32, (te, 8), 1) == cat_ref[...]).astype(jnp.float32)
    ef16 = jnp.dot(oh, te_ref[...], preferred_element_type=jnp.float32,
                   precision=lax.Precision.HIGHEST)
    hp = lax.Precision.HIGHEST
    a = (jnp.dot(ef16, w1_ref[0:16, :], preferred_element_type=jnp.float32, precision=hp)
         + jnp.dot(scal_ref[...], w1_ref[16:20, :], preferred_element_type=jnp.float32, precision=hp)
         + b1_ref[...])
    a = jnp.maximum(a, 0.0)
    a_hi = a.astype(jnp.bfloat16)
    a_lo = (a - a_hi.astype(jnp.float32)).astype(jnp.bfloat16)
    ew = (jnp.dot(a_hi, w2h_ref[...], preferred_element_type=jnp.float32)
          + jnp.dot(a_lo, w2h_ref[...], preferred_element_type=jnp.float32)
          + jnp.dot(a_hi, w2l_ref[...], preferred_element_type=jnp.float32)
          + b2_ref[...])
    egl = pid * te + lax.broadcasted_iota(jnp.int32, (te, 1), 0)
    out_ref[...] = jnp.where(egl < e_real, ew, 0.0)


def _edge_weights(cat_p, scal_p, tbl, w1, b1, w2, b2, e_real):
    w2h = w2.astype(jnp.bfloat16)
    w2l = (w2 - w2h.astype(jnp.float32)).astype(jnp.bfloat16)
    e_pad = cat_p.shape[0]
    te = 512
    grid = e_pad // te
    return pl.pallas_call(
        functools.partial(_edge_kernel, te=te, e_real=e_real),
        grid=(grid,),
        in_specs=[
            pl.BlockSpec((te, 1), lambda i: (i, 0)),
            pl.BlockSpec((te, 4), lambda i: (i, 0)),
            pl.BlockSpec((8, 16), lambda i: (0, 0)),
            pl.BlockSpec((20, 128), lambda i: (0, 0)),
            pl.BlockSpec((1, 128), lambda i: (0, 0)),
            pl.BlockSpec((128, D * D), lambda i: (0, 0)),
            pl.BlockSpec((128, D * D), lambda i: (0, 0)),
            pl.BlockSpec((1, D * D), lambda i: (0, 0)),
        ],
        out_specs=pl.BlockSpec((te, D * D), lambda i: (i, 0)),
        out_shape=jax.ShapeDtypeStruct((e_pad, D * D), jnp.float32),
    )(cat_p, scal_p, tbl[0], w1, b1.reshape(1, 128), w2h, w2l, b2.reshape(1, D * D))


def _msg_kernel(h_ref, ew_ref, r_ref, s_ref, out_ref, *, tm):
    h4 = jnp.dot(h_ref[...], r_ref[...], preferred_element_type=jnp.float32)
    p = h4 * ew_ref[...]
    out_ref[...] = jnp.dot(p, s_ref[...], preferred_element_type=jnp.float32)


def _messages(hsrc, ew2d, rmat, smat):
    e_pad = hsrc.shape[0]
    tm = 512
    return pl.pallas_call(
        functools.partial(_msg_kernel, tm=tm),
        grid=(e_pad // tm,),
        in_specs=[
            pl.BlockSpec((tm, D), lambda i: (i, 0)),
            pl.BlockSpec((tm, D * D), lambda i: (i, 0)),
            pl.BlockSpec((D, D * D), lambda i: (0, 0)),
            pl.BlockSpec((D * D, D), lambda i: (0, 0)),
        ],
        out_specs=pl.BlockSpec((tm, D), lambda i: (i, 0)),
        out_shape=jax.ShapeDtypeStruct((e_pad, D), jnp.float32),
    )(hsrc, ew2d, rmat, smat)


def _dot3(x, wh, wl):
    x_hi = x.astype(jnp.bfloat16)
    x_lo = (x - x_hi.astype(jnp.float32)).astype(jnp.bfloat16)
    return (jnp.dot(x_hi, wh, preferred_element_type=jnp.float32)
            + jnp.dot(x_lo, wh, preferred_element_type=jnp.float32)
            + jnp.dot(x_hi, wl, preferred_element_type=jnp.float32))


def _gru_kernel(agg_ref, hid_ref, nnb_ref, wih_ref, wil_ref, bi_ref, whh_ref,
                whl_ref, bh_ref, out_ref):
    agg = agg_ref[0] + agg_ref[1]
    x = jnp.maximum(agg + nnb_ref[...], 0.0)
    h = hid_ref[...]
    gi = _dot3(x, wih_ref[...], wil_ref[...]) + bi_ref[...]
    gh = _dot3(h, whh_ref[...], whl_ref[...]) + bh_ref[...]
    r = jax.nn.sigmoid(gi[:, 0:D] + gh[:, 0:D])
    z = jax.nn.sigmoid(gi[:, D:2 * D] + gh[:, D:2 * D])
    nn = jnp.tanh(gi[:, 2 * D:3 * D] + r * gh[:, 2 * D:3 * D])
    out_ref[...] = (1.0 - z) * nn + z * h


def _split3(w):
    wh = w.astype(jnp.bfloat16)
    wl = (w - wh.astype(jnp.float32)).astype(jnp.bfloat16)
    return wh, wl


def _gru(agg2, hidden, nnb, wi, bi, wh, bh):
    n = hidden.shape[0]
    wih, wil = _split3(wi)
    whh, whl = _split3(wh)
    return pl.pallas_call(
        _gru_kernel,
        out_shape=jax.ShapeDtypeStruct((n, D), jnp.float32),
    )(agg2, hidden, nnb.reshape(1, D), wih, wil, bi.reshape(1, 3 * D),
      whh, whl, bh.reshape(1, 3 * D))


def _readout_kernel(h_ref, grow_ref, gcol_ref, w_ref, b_ref, out_ref, *, n):
    h = h_ref[...]
    w = jax.nn.sigmoid(jnp.dot(h, w_ref[...], preferred_element_type=jnp.float32,
                               precision=lax.Precision.HIGHEST) + b_ref[...])
    wh = w * h
    onehot = (grow_ref[...] == lax.broadcasted_iota(jnp.int32, (B, n), 0)).astype(jnp.float32)
    ws = jnp.dot(onehot, wh, preferred_element_type=jnp.float32,
                 precision=lax.Precision.HIGHEST)
    gcol = gcol_ref[...]
    mxs = []
    for bb in range(B):
        hm = jnp.where(gcol == bb, h, -jnp.inf)
        mxs.append(jnp.max(hm, axis=0, keepdims=True))
    mx = jnp.concatenate(mxs, axis=0)
    out_ref[...] = jnp.concatenate([ws, mx], axis=1)


def _readout(h, gids, w, b):
    n = h.shape[0]
    return pl.pallas_call(
        functools.partial(_readout_kernel, n=n),
        out_shape=jax.ShapeDtypeStruct((B, 2 * D), jnp.float32),
    )(h, gids.reshape(1, n), gids.reshape(n, 1), w, b.reshape(1, 1))


def _head_kernel(rr_ref, lr_ref, w1_ref, b1_ref, g1_ref, bb1_ref,
                 w2_ref, b2_ref, g2_ref, bb2_ref, ow_ref, ob_ref, out_ref):
    rr = rr_ref[...]
    lr = lr_ref[...]
    C = jnp.dot(lr, w1_ref[...], preferred_element_type=jnp.float32,
                precision=lax.Precision.HIGHEST)
    y = jnp.zeros((B, 256), jnp.float32)
    for i in range(64):
        y = y + rr[:, i:i + 1] * C[:, i * 256:(i + 1) * 256]
    y = y + b1_ref[...]
    mu = jnp.mean(y, axis=-1, keepdims=True)
    v = jnp.mean((y - mu) * (y - mu), axis=-1, keepdims=True)
    y = (y - mu) * lax.rsqrt(v + 1e-5) * g1_ref[...] + bb1_ref[...]
    y = jnp.where(y > 0, y, 0.01 * y)
    y = jnp.dot(y, w2_ref[...], preferred_element_type=jnp.float32,
                precision=lax.Precision.HIGHEST) + b2_ref[...]
    mu = jnp.mean(y, axis=-1, keepdims=True)
    v = jnp.mean((y - mu) * (y - mu), axis=-1, keepdims=True)
    y = (y - mu) * lax.rsqrt(v + 1e-5) * g2_ref[...] + bb2_ref[...]
    y = jnp.where(y > 0, y, 0.01 * y)
    out_ref[...] = jnp.dot(y, ow_ref[...], preferred_element_type=jnp.float32,
                           precision=lax.Precision.HIGHEST) + ob_ref[...]


def _head(rr, lr, p):
    w1v = p['mlp_W1'].reshape(64, 64, 256).transpose(1, 0, 2).reshape(64, 64 * 256)
    out = pl.pallas_call(
        _head_kernel,
        out_shape=jax.ShapeDtypeStruct((B, 1), jnp.float32),
    )(rr, lr, w1v, p['mlp_b1'].reshape(1, 256), p['ln1_g'].reshape(1, 256),
      p['ln1_b'].reshape(1, 256), p['mlp_W2'], p['mlp_b2'].reshape(1, 64),
      p['ln2_g'].reshape(1, 64), p['ln2_b'].reshape(1, 64), p['out_W'],
      p['out_b'].reshape(1, 1))
    return out[:, 0]


def _sc_gather(h, src3):
    nw, ch, ck = src3.shape
    eb = ch * ck
    e_pad = nw * eb

    @functools.partial(
        pl.kernel,
        out_type=jax.ShapeDtypeStruct((e_pad, D), jnp.float32),
        mesh=_sc_mesh(),
        compiler_params=pltpu.CompilerParams(use_tc_tiling_on_sc=False),
        scratch_types=[
            pltpu.VMEM((ch, ck), jnp.int32),
            pltpu.VMEM((eb, D), jnp.float32),
            pltpu.SemaphoreType.DMA,
        ],
    )
    def gk(h_hbm, src_hbm, out_hbm, idx_v, rows_v, sem):
        wid = lax.axis_index("s") * 2 + lax.axis_index("c")
        base = wid * eb
        pltpu.sync_copy(src_hbm.at[wid], idx_v)
        for j in range(ch):
            pltpu.async_copy(h_hbm.at[idx_v.at[j]], rows_v.at[pl.ds(j * ck, ck)], sem)
        for j in range(ch):
            pltpu.make_async_copy(h_hbm.at[idx_v.at[j]],
                                  rows_v.at[pl.ds(j * ck, ck)], sem).wait()
        pltpu.sync_copy(rows_v, out_hbm.at[pl.ds(base, eb)])

    return gk(h, src3)


def _sc_scatter(msg, dst3, zeros_n):
    nw, ch, ck = dst3.shape
    eb = ch * ck
    n = zeros_n.shape[0]
    nr = n // 16

    @functools.partial(
        pl.kernel,
        out_type=jax.ShapeDtypeStruct((2, n, D), jnp.float32),
        mesh=_sc_mesh(),
        compiler_params=pltpu.CompilerParams(use_tc_tiling_on_sc=False),
        scratch_types=[
            pltpu.VMEM((ch, ck), jnp.int32),
            pltpu.VMEM((eb, D), jnp.float32),
            pltpu.VMEM_SHARED((n, D), jnp.float32),
        ],
    )
    def sk(msg_hbm, dst_hbm, zeros_hbm, out_hbm, idx_v, msg_v, acc_sh):
        c = lax.axis_index("c")
        s = lax.axis_index("s")
        wid = s * 2 + c
        base = wid * eb
        pltpu.sync_copy(dst_hbm.at[wid], idx_v)
        pltpu.sync_copy(msg_hbm.at[pl.ds(base, eb)], msg_v)
        pltpu.sync_copy(zeros_hbm.at[pl.ds(s * nr, nr)], acc_sh.at[pl.ds(s * nr, nr)])
        plsc.subcore_barrier()
        for j in range(ch):
            pltpu.sync_copy(msg_v.at[pl.ds(j * ck, ck)], acc_sh.at[idx_v.at[j]], add=True)
        plsc.subcore_barrier()
        pltpu.sync_copy(acc_sh.at[pl.ds(s * nr, nr)], out_hbm.at[c, pl.ds(s * nr, nr)])

    return sk(msg, dst3, zeros_n)


def _mpnn(p, pre, cat, scal, ecat_p, escal_p, src_p, dst_p, n_nodes, e_real, n_layers):
    h = _node_embed_proj(cat, scal, p[pre + '_node_emb'], p[pre + '_proj_W'], p[pre + '_proj_b'])
    ew = _edge_weights(ecat_p, escal_p, p[pre + '_edge_emb'], p[pre + '_enW1'],
                       p[pre + '_enb1'], p[pre + '_enW2'], p[pre + '_enb2'], e_real)
    eye = jnp.eye(D, dtype=jnp.float32)
    rmat = jnp.kron(eye, jnp.ones((1, D), jnp.float32))
    smat = jnp.tile(eye, (D, 1))
    zeros_n = jnp.zeros((n_nodes, D), jnp.float32)
    eb = src_p.shape[0] // NW
    ck = 128 if eb % 128 == 0 else eb
    src3 = src_p.reshape(NW, eb // ck, ck)
    dst3 = dst_p.reshape(NW, eb // ck, ck)
    hidden = h
    for _ in range(n_layers):
        hsrc = _sc_gather(h, src3)
        msg = _messages(hsrc, ew, rmat, smat)
        agg2 = _sc_scatter(msg, dst3, zeros_n)
        hidden = _gru(agg2, hidden, p[pre + '_nn_b'], p[pre + '_Wi'],
                      p[pre + '_bi'], p[pre + '_Wh'], p[pre + '_bh'])
        h = hidden
    return h


def _pad_edges(arr, e_pad):
    e = arr.shape[0]
    if e == e_pad:
        return arr
    pad = [(0, e_pad - e)] + [(0, 0)] * (arr.ndim - 1)
    return jnp.pad(arr, pad)


@jax.jit
def kernel(rec_node_cat, rec_node_scal, rec_edge_cat, rec_edge_scal, rec_edge_src, rec_edge_dst, rec_graph_ids, lig_node_cat, lig_node_scal, lig_edge_cat, lig_edge_scal, lig_edge_src, lig_edge_dst, lig_graph_ids, params):
    p = params
    rec_ep = 40960
    lig_ep = LIG_E

    rec_out = _mpnn(p, 'rec', rec_node_cat, rec_node_scal,
                    _pad_edges(rec_edge_cat, rec_ep), _pad_edges(rec_edge_scal, rec_ep),
                    _pad_edges(rec_edge_src, rec_ep), _pad_edges(rec_edge_dst, rec_ep),
                    REC_N, REC_E, 2)
    lig_out = _mpnn(p, 'lig', lig_node_cat, lig_node_scal,
                    lig_edge_cat, lig_edge_scal, lig_edge_src, lig_edge_dst,
                    LIG_N, LIG_E, 3)
    rr = _readout(rec_out, rec_graph_ids, p['rec_rw_W'], p['rec_rw_b'])
    lr = _readout(lig_out, lig_graph_ids, p['lig_rw_W'], p['lig_rw_b'])
    return _head(rr, lr, p)
